# baseline (speedup 1.0000x reference)
import functools

import numpy as np
import jax
import jax.numpy as jnp
from jax import lax
from jax.experimental import pallas as pl
from jax.experimental.pallas import tpu as pltpu
from jax.experimental.pallas import tpu_sc as plsc

_N = 2048
_NOISE_DIM = 128
_CLASS_EMBED_DIM = 64
_HIDDEN_DIM = 512
_NODE_FEAT_DIM = 256
_NUM_CLASSES = 10

_M = _N * (_N - 1) // 2
_NW = 32
_SEG = _M // _NW
_KBUF = 12
_ROWP = 2064
_BLK = 256

_iu_np, _ju_np = np.triu_indices(_N, k=1)
_PAIR_NP = np.stack([_iu_np, _ju_np], axis=0).astype(np.int32)


def _pair_copy_body(pair_in_ref, pair_out_ref):
    pair_out_ref[...] = pair_in_ref[...]


_OFF_NP = (np.arange(_N + 1, dtype=np.int64) *
           (2 * _N - 1 - np.arange(_N + 1, dtype=np.int64))) // 2
_ROW_COST = 140
_TOT_COST = _M + _ROW_COST * _N
_A0S = []
for _w in range(_NW):
    _tgt = _w * _TOT_COST / _NW
    _p = np.searchsorted(
        np.arange(0, _M, 8) + _ROW_COST * (
            np.searchsorted(_OFF_NP, np.arange(0, _M, 8), side="right") - 1),
        _tgt)
    _A0S.append(int(min(_p, _M // 8 - 1)) * 8)
_A0S.append(_M)
_SEGL = [_A0S[w + 1] - _A0S[w] for w in range(_NW)]
_SEG_MAX = max(_SEGL)
_R0S = [int(np.searchsorted(_OFF_NP, _A0S[w], side="right") - 1)
        for w in range(_NW)]
_R1S = [int(np.searchsorted(_OFF_NP, _A0S[w + 1] - 1, side="right"))
        for w in range(_NW)]
_WCLS = (2048, 1024, 512)
_CLS = [max(c for c, wdt in enumerate(_WCLS) if wdt >= 2047 - _R0S[w])
        for w in range(_NW)]


def _gen_body(labels_ref, thr_ref, ctab_ref, z_ref, w1_ref, b1_ref,
              w2_ref, b2_ref, we_ref, be_ref, out_ref, h_scr):
    i = pl.program_id(0)

    @pl.when(i == 0)
    def _():
        lab = labels_ref[0]
        sel = (lax.broadcasted_iota(jnp.int32, (_NUM_CLASSES, 1), 0) == lab)
        ce = jnp.sum(jnp.where(sel, ctab_ref[...], 0.0), axis=0, keepdims=True)
        b1_eff = b1_ref[...] + jnp.dot(ce, w1_ref[pl.ds(_NOISE_DIM, _CLASS_EMBED_DIM), :],
                                       preferred_element_type=jnp.float32)
        hgen = jnp.maximum(
            jnp.dot(z_ref[...], w1_ref[pl.ds(0, _NOISE_DIM), :],
                    preferred_element_type=jnp.float32) + b1_eff, 0.0)
        x = jnp.dot(hgen, w2_ref[...],
                    preferred_element_type=jnp.float32) + b2_ref[...]
        h_scr[...] = jnp.maximum(
            jnp.dot(x, we_ref[...],
                    preferred_element_type=jnp.float32) + be_ref[...], 0.0)

    hi = h_scr[pl.ds(i * _BLK, _BLK), :]
    h = h_scr[...]
    g = lax.dot_general(hi, h, (((1,), (1,)), ((), ())),
                        preferred_element_type=jnp.float32)
    sq_i = jnp.sum(hi * hi, axis=1, keepdims=True)
    sq_j = lax.dot_general(jnp.ones((1, _HIDDEN_DIM), jnp.float32), h * h,
                           (((1,), (1,)), ((), ())),
                           preferred_element_type=jnp.float32)
    d2 = sq_i + sq_j - 2.0 * g
    dist = jnp.sqrt(jnp.clip(d2, 1e-12, None))
    probs = jax.nn.sigmoid(thr_ref[0] - dist)
    rows = i * _BLK + lax.broadcasted_iota(jnp.int32, (_BLK, _N), 0)
    cols = lax.broadcasted_iota(jnp.int32, (_BLK, _N), 1)
    out_ref[...] = jnp.where(rows == cols, 0.0, probs)


def _tri_off(i):
    return (i * (2 * _N - 1 - i)) // 2


@functools.cache
def _make_triu_gather():
    @functools.partial(
        pl.kernel,
        out_type=jax.ShapeDtypeStruct((_M,), jnp.float32),
        mesh=plsc.VectorSubcoreMesh(core_axis_name="c", subcore_axis_name="s"),
        scratch_types=[
            pltpu.VMEM((2 * _KBUF * _ROWP,), jnp.float32),
            pltpu.VMEM((_SEG_MAX + _ROWP,), jnp.float32),
            [pltpu.SemaphoreType.DMA] * (2 * _KBUF),
        ],
    )
    def _triu_gather(adj_hbm, out_hbm, rows_v, seg_v, sems):
        cid = lax.axis_index("c")
        sid = lax.axis_index("s")
        wid = sid * 2 + cid

        a0 = jnp.int32(_A0S[0])
        seg_len = jnp.int32(_SEGL[0])
        r0 = jnp.int32(_R0S[0])
        r1 = jnp.int32(_R1S[0])
        cls = jnp.int32(_CLS[0])
        for w in range(1, _NW):
            a0 = jnp.where(wid == w, jnp.int32(_A0S[w]), a0)
            seg_len = jnp.where(wid == w, jnp.int32(_SEGL[w]), seg_len)
            r0 = jnp.where(wid == w, jnp.int32(_R0S[w]), r0)
            r1 = jnp.where(wid == w, jnp.int32(_R1S[w]), r1)
            cls = jnp.where(wid == w, jnp.int32(_CLS[w]), cls)
        a0 = pl.multiple_of(a0, 8)
        seg_len = pl.multiple_of(seg_len, 8)
        nrows = r1 - r0
        wsel = jnp.int32(_WCLS[0])
        for c in range(1, len(_WCLS)):
            wsel = jnp.where(cls == c, jnp.int32(_WCLS[c]), wsel)
        ngroups = (nrows + _KBUF - 1) // _KBUF

        def fire(t, bank):
            rbase = r0 + t * _KBUF
            for b in range(_KBUF):
                i = rbase + b
                slot = bank * _KBUF + b
                live = (t < ngroups) & (i < r1)
                for c, wdt in enumerate(_WCLS):

                    @pl.when(live & (cls == c))
                    def _(i=i, slot=slot, wdt=wdt):
                        pltpu.async_copy(
                            adj_hbm.at[pl.ds(i * _N + (_N - wdt), wdt)],
                            rows_v.at[pl.ds(slot * _ROWP, wdt)],
                            sems[slot])

        def process(t, bank):
            rbase = r0 + t * _KBUF
            for b in range(_KBUF):
                i = rbase + b
                slot = bank * _KBUF + b
                live = (t < ngroups) & (i < r1)
                for c, wdt in enumerate(_WCLS):

                    @pl.when(live & (cls == c))
                    def _(i=i, slot=slot, wdt=wdt):
                        pltpu.make_async_copy(
                            adj_hbm.at[pl.ds(i * _N + (_N - wdt), wdt)],
                            rows_v.at[pl.ds(slot * _ROWP, wdt)],
                            sems[slot]).wait()

                @pl.when(live)
                def _(i=i, slot=slot):
                    off_i = _tri_off(i)
                    skip = jnp.maximum(a0 - off_i, 0)
                    col0 = i + 1 + skip
                    q = off_i + skip - a0
                    length = (_N - 1 - i) - skip
                    nv = (length + 15) >> 4
                    src0 = slot * _ROWP + col0 - _N + wsel

                    def copy16(u, carry):
                        seg_v[pl.ds(q + u * 16, 16)] = (
                            rows_v[pl.ds(src0 + u * 16, 16)])
                        return carry

                    lax.fori_loop(0, nv, copy16, 0)

        fire(0, 0)

        def two_groups(tt, carry):
            t0 = 2 * tt
            fire(t0 + 1, 1)
            process(t0, 0)
            fire(t0 + 2, 0)
            process(t0 + 1, 1)
            return carry

        lax.fori_loop(0, (ngroups + 1) // 2, two_groups, 0)
        _bits = range(16, 2, -1)
        pos = jnp.int32(0)
        for n, p in enumerate(_bits):
            on = ((seg_len >> p) & 1) == 1

            @pl.when(on)
            def _(p=p, pos=pos, n=n):
                src = pl.multiple_of(pos, 8)
                dst = pl.multiple_of(a0 + pos, 8)
                pltpu.async_copy(seg_v.at[pl.ds(src, 1 << p)],
                                 out_hbm.at[pl.ds(dst, 1 << p)],
                                 sems[n])

            pos = pos + jnp.where(on, jnp.int32(1 << p), 0)
        pos = jnp.int32(0)
        for n, p in enumerate(_bits):
            on = ((seg_len >> p) & 1) == 1

            @pl.when(on)
            def _(p=p, pos=pos, n=n):
                src = pl.multiple_of(pos, 8)
                dst = pl.multiple_of(a0 + pos, 8)
                pltpu.make_async_copy(seg_v.at[pl.ds(src, 1 << p)],
                                      out_hbm.at[pl.ds(dst, 1 << p)],
                                      sems[n]).wait()

            pos = pos + jnp.where(on, jnp.int32(1 << p), 0)

    return _triu_gather


def kernel(class_labels, z, class_table, Wg1, bg1, Wg2, bg2, We, be, threshold):
    nblk = _N // _BLK
    soft_adj = pl.pallas_call(
        _gen_body,
        grid=(nblk,),
        out_shape=jax.ShapeDtypeStruct((_N, _N), jnp.float32),
        in_specs=[
            pl.BlockSpec(memory_space=pltpu.SMEM),
            pl.BlockSpec(memory_space=pltpu.SMEM),
            pl.BlockSpec(memory_space=pltpu.VMEM),
            pl.BlockSpec(memory_space=pltpu.VMEM),
            pl.BlockSpec(memory_space=pltpu.VMEM),
            pl.BlockSpec(memory_space=pltpu.VMEM),
            pl.BlockSpec(memory_space=pltpu.VMEM),
            pl.BlockSpec(memory_space=pltpu.VMEM),
            pl.BlockSpec(memory_space=pltpu.VMEM),
            pl.BlockSpec(memory_space=pltpu.VMEM),
        ],
        out_specs=pl.BlockSpec((_BLK, _N), lambda i: (i, 0)),
        scratch_shapes=[pltpu.VMEM((_N, _HIDDEN_DIM), jnp.float32)],
    )(class_labels, jnp.reshape(threshold, (1,)), class_table, z, Wg1,
      bg1[None, :], Wg2, bg2[None, :], We, be[None, :])

    probs_flat = _make_triu_gather()(jnp.reshape(soft_adj, (_N * _N,)))
    probs = probs_flat[:, None]

    pair_cols = _M // nblk
    pair_index = pl.pallas_call(
        _pair_copy_body,
        grid=(nblk,),
        out_shape=jax.ShapeDtypeStruct((2, _M), jnp.int32),
        in_specs=[pl.BlockSpec((2, pair_cols), lambda i: (0, i))],
        out_specs=pl.BlockSpec((2, pair_cols), lambda i: (0, i)),
    )(jnp.asarray(_PAIR_NP))
    return probs, pair_index, soft_adj


# speedup vs baseline: 1.0328x; 1.0328x over previous
import functools

import numpy as np
import jax
import jax.numpy as jnp
from jax import lax
from jax.experimental import pallas as pl
from jax.experimental.pallas import tpu as pltpu
from jax.experimental.pallas import tpu_sc as plsc

_N = 2048
_NOISE_DIM = 128
_CLASS_EMBED_DIM = 64
_HIDDEN_DIM = 512
_NODE_FEAT_DIM = 256
_NUM_CLASSES = 10

_M = _N * (_N - 1) // 2
_NW = 32
_SEG = _M // _NW
_KBUF = 8
_ROWP = 2064
_BLK = 256

_iu_np, _ju_np = np.triu_indices(_N, k=1)
_PAIR_NP = np.stack([_iu_np, _ju_np], axis=0).astype(np.int32)


def _pair_copy_body(pair_in_ref, pair_out_ref):
    pair_out_ref[...] = pair_in_ref[...]


_OFF_NP = (np.arange(_N + 1, dtype=np.int64) *
           (2 * _N - 1 - np.arange(_N + 1, dtype=np.int64))) // 2
_ROW_COST = 140
_TOT_COST = _M + _ROW_COST * _N
_A0S = []
for _w in range(_NW):
    _tgt = _w * _TOT_COST / _NW
    _p = np.searchsorted(
        np.arange(0, _M, 8) + _ROW_COST * (
            np.searchsorted(_OFF_NP, np.arange(0, _M, 8), side="right") - 1),
        _tgt)
    _A0S.append(int(min(_p, _M // 8 - 1)) * 8)
_A0S.append(_M)
_SEGL = [_A0S[w + 1] - _A0S[w] for w in range(_NW)]
_SEG_MAX = max(_SEGL)
_R0S = [int(np.searchsorted(_OFF_NP, _A0S[w], side="right") - 1)
        for w in range(_NW)]
_R1S = [int(np.searchsorted(_OFF_NP, _A0S[w + 1] - 1, side="right"))
        for w in range(_NW)]
_WCLS = (2048, 1024, 512)
_CLS = [max(c for c, wdt in enumerate(_WCLS) if wdt >= 2047 - _R0S[w])
        for w in range(_NW)]


def _gen_body(labels_ref, thr_ref, ctab_ref, z_ref, w1_ref, b1_ref,
              w2_ref, b2_ref, we_ref, be_ref, out_ref, h_scr):
    i = pl.program_id(0)

    @pl.when(i == 0)
    def _():
        lab = labels_ref[0]
        sel = (lax.broadcasted_iota(jnp.int32, (_NUM_CLASSES, 1), 0) == lab)
        ce = jnp.sum(jnp.where(sel, ctab_ref[...], 0.0), axis=0, keepdims=True)
        b1_eff = b1_ref[...] + jnp.dot(ce, w1_ref[pl.ds(_NOISE_DIM, _CLASS_EMBED_DIM), :],
                                       preferred_element_type=jnp.float32)
        hgen = jnp.maximum(
            jnp.dot(z_ref[...], w1_ref[pl.ds(0, _NOISE_DIM), :],
                    preferred_element_type=jnp.float32) + b1_eff, 0.0)
        x = jnp.dot(hgen, w2_ref[...],
                    preferred_element_type=jnp.float32) + b2_ref[...]
        h_scr[...] = jnp.maximum(
            jnp.dot(x, we_ref[...],
                    preferred_element_type=jnp.float32) + be_ref[...], 0.0)

    hi = h_scr[pl.ds(i * _BLK, _BLK), :]
    h = h_scr[...]
    g = lax.dot_general(hi, h, (((1,), (1,)), ((), ())),
                        preferred_element_type=jnp.float32)
    sq_i = jnp.sum(hi * hi, axis=1, keepdims=True)
    sq_j = lax.dot_general(jnp.ones((1, _HIDDEN_DIM), jnp.float32), h * h,
                           (((1,), (1,)), ((), ())),
                           preferred_element_type=jnp.float32)
    d2 = sq_i + sq_j - 2.0 * g
    dist = jnp.sqrt(jnp.clip(d2, 1e-12, None))
    probs = jax.nn.sigmoid(thr_ref[0] - dist)
    rows = i * _BLK + lax.broadcasted_iota(jnp.int32, (_BLK, _N), 0)
    cols = lax.broadcasted_iota(jnp.int32, (_BLK, _N), 1)
    out_ref[...] = jnp.where(rows == cols, 0.0, probs)


def _tri_off(i):
    return (i * (2 * _N - 1 - i)) // 2


@functools.cache
def _make_triu_gather():
    @functools.partial(
        pl.kernel,
        out_type=jax.ShapeDtypeStruct((_M,), jnp.float32),
        mesh=plsc.VectorSubcoreMesh(core_axis_name="c", subcore_axis_name="s"),
        scratch_types=[
            pltpu.VMEM((2 * _KBUF * _ROWP,), jnp.float32),
            pltpu.VMEM((_SEG_MAX + _ROWP,), jnp.float32),
            [pltpu.SemaphoreType.DMA] * (2 * _KBUF),
        ],
    )
    def _triu_gather(adj_hbm, out_hbm, rows_v, seg_v, sems):
        cid = lax.axis_index("c")
        sid = lax.axis_index("s")
        wid = sid * 2 + cid

        a0 = jnp.int32(_A0S[0])
        seg_len = jnp.int32(_SEGL[0])
        r0 = jnp.int32(_R0S[0])
        r1 = jnp.int32(_R1S[0])
        cls = jnp.int32(_CLS[0])
        for w in range(1, _NW):
            a0 = jnp.where(wid == w, jnp.int32(_A0S[w]), a0)
            seg_len = jnp.where(wid == w, jnp.int32(_SEGL[w]), seg_len)
            r0 = jnp.where(wid == w, jnp.int32(_R0S[w]), r0)
            r1 = jnp.where(wid == w, jnp.int32(_R1S[w]), r1)
            cls = jnp.where(wid == w, jnp.int32(_CLS[w]), cls)
        a0 = pl.multiple_of(a0, 8)
        seg_len = pl.multiple_of(seg_len, 8)
        nrows = r1 - r0
        wsel = jnp.int32(_WCLS[0])
        for c in range(1, len(_WCLS)):
            wsel = jnp.where(cls == c, jnp.int32(_WCLS[c]), wsel)
        ngroups = (nrows + _KBUF - 1) // _KBUF

        def fire(t, bank):
            rbase = r0 + t * _KBUF
            for b in range(_KBUF):
                i = rbase + b
                slot = bank * _KBUF + b
                live = (t < ngroups) & (i < r1)
                for c, wdt in enumerate(_WCLS):

                    @pl.when(live & (cls == c))
                    def _(i=i, slot=slot, wdt=wdt):
                        pltpu.async_copy(
                            adj_hbm.at[pl.ds(i * _N + (_N - wdt), wdt)],
                            rows_v.at[pl.ds(slot * _ROWP, wdt)],
                            sems[slot])

        def process(t, bank):
            rbase = r0 + t * _KBUF
            for b in range(_KBUF):
                i = rbase + b
                slot = bank * _KBUF + b
                live = (t < ngroups) & (i < r1)
                for c, wdt in enumerate(_WCLS):

                    @pl.when(live & (cls == c))
                    def _(i=i, slot=slot, wdt=wdt):
                        pltpu.make_async_copy(
                            adj_hbm.at[pl.ds(i * _N + (_N - wdt), wdt)],
                            rows_v.at[pl.ds(slot * _ROWP, wdt)],
                            sems[slot]).wait()

                @pl.when(live)
                def _(i=i, slot=slot):
                    off_i = _tri_off(i)
                    skip = jnp.maximum(a0 - off_i, 0)
                    col0 = i + 1 + skip
                    q = off_i + skip - a0
                    length = (_N - 1 - i) - skip
                    nv = (length + 15) >> 4
                    src0 = slot * _ROWP + col0 - _N + wsel

                    def copy16(u, carry):
                        seg_v[pl.ds(q + u * 16, 16)] = (
                            rows_v[pl.ds(src0 + u * 16, 16)])
                        return carry

                    lax.fori_loop(0, nv, copy16, 0)

        fire(0, 0)

        def two_groups(tt, carry):
            t0 = 2 * tt
            fire(t0 + 1, 1)
            process(t0, 0)
            fire(t0 + 2, 0)
            process(t0 + 1, 1)
            return carry

        lax.fori_loop(0, (ngroups + 1) // 2, two_groups, 0)
        _bits = range(16, 2, -1)
        pos = jnp.int32(0)
        for n, p in enumerate(_bits):
            on = ((seg_len >> p) & 1) == 1

            @pl.when(on)
            def _(p=p, pos=pos, n=n):
                src = pl.multiple_of(pos, 8)
                dst = pl.multiple_of(a0 + pos, 8)
                pltpu.async_copy(seg_v.at[pl.ds(src, 1 << p)],
                                 out_hbm.at[pl.ds(dst, 1 << p)],
                                 sems[n])

            pos = pos + jnp.where(on, jnp.int32(1 << p), 0)
        pos = jnp.int32(0)
        for n, p in enumerate(_bits):
            on = ((seg_len >> p) & 1) == 1

            @pl.when(on)
            def _(p=p, pos=pos, n=n):
                src = pl.multiple_of(pos, 8)
                dst = pl.multiple_of(a0 + pos, 8)
                pltpu.make_async_copy(seg_v.at[pl.ds(src, 1 << p)],
                                      out_hbm.at[pl.ds(dst, 1 << p)],
                                      sems[n]).wait()

            pos = pos + jnp.where(on, jnp.int32(1 << p), 0)

    return _triu_gather


def kernel(class_labels, z, class_table, Wg1, bg1, Wg2, bg2, We, be, threshold):
    nblk = _N // _BLK
    soft_adj = pl.pallas_call(
        _gen_body,
        grid=(nblk,),
        out_shape=jax.ShapeDtypeStruct((_N, _N), jnp.float32),
        in_specs=[
            pl.BlockSpec(memory_space=pltpu.SMEM),
            pl.BlockSpec(memory_space=pltpu.SMEM),
            pl.BlockSpec(memory_space=pltpu.VMEM),
            pl.BlockSpec(memory_space=pltpu.VMEM),
            pl.BlockSpec(memory_space=pltpu.VMEM),
            pl.BlockSpec(memory_space=pltpu.VMEM),
            pl.BlockSpec(memory_space=pltpu.VMEM),
            pl.BlockSpec(memory_space=pltpu.VMEM),
            pl.BlockSpec(memory_space=pltpu.VMEM),
            pl.BlockSpec(memory_space=pltpu.VMEM),
        ],
        out_specs=pl.BlockSpec((_BLK, _N), lambda i: (i, 0)),
        scratch_shapes=[pltpu.VMEM((_N, _HIDDEN_DIM), jnp.float32)],
    )(class_labels, jnp.reshape(threshold, (1,)), class_table, z, Wg1,
      bg1[None, :], Wg2, bg2[None, :], We, be[None, :])

    probs_flat = _make_triu_gather()(jnp.reshape(soft_adj, (_N * _N,)))
    probs = probs_flat[:, None]

    pair_cols = _M // nblk
    pair_index = pl.pallas_call(
        _pair_copy_body,
        grid=(nblk,),
        out_shape=jax.ShapeDtypeStruct((2, _M), jnp.int32),
        in_specs=[pl.BlockSpec((2, pair_cols), lambda i: (0, i))],
        out_specs=pl.BlockSpec((2, pair_cols), lambda i: (0, i)),
    )(jnp.asarray(_PAIR_NP))
    return probs, pair_index, soft_adj


# speedup vs baseline: 1.0351x; 1.0023x over previous
import functools

import numpy as np
import jax
import jax.numpy as jnp
from jax import lax
from jax.experimental import pallas as pl
from jax.experimental.pallas import tpu as pltpu
from jax.experimental.pallas import tpu_sc as plsc

_N = 2048
_NOISE_DIM = 128
_CLASS_EMBED_DIM = 64
_HIDDEN_DIM = 512
_NODE_FEAT_DIM = 256
_NUM_CLASSES = 10

_M = _N * (_N - 1) // 2
_NW = 32
_KBUF = 8
_ROWP = 2064
_BLK = 256

_iu_np, _ju_np = np.triu_indices(_N, k=1)
_PAIR_NP = np.stack([_iu_np, _ju_np], axis=0).astype(np.int32)


def _pair_copy_body(pair_in_ref, pair_out_ref):
    pair_out_ref[...] = pair_in_ref[...]


_OFF_NP = (np.arange(_N + 1, dtype=np.int64) *
           (2 * _N - 1 - np.arange(_N + 1, dtype=np.int64))) // 2
_ROW_COST = 140
_TOT_COST = _M + _ROW_COST * _N
_A0S = []
for _w in range(_NW):
    _tgt = _w * _TOT_COST / _NW
    _p = np.searchsorted(
        np.arange(0, _M, 8) + _ROW_COST * (
            np.searchsorted(_OFF_NP, np.arange(0, _M, 8), side="right") - 1),
        _tgt)
    _A0S.append(int(min(_p, _M // 8 - 1)) * 8)
_A0S.append(_M)
_SEGL = [_A0S[w + 1] - _A0S[w] for w in range(_NW)]
_SEG_MAX = max(_SEGL)
_R0S = [int(np.searchsorted(_OFF_NP, _A0S[w], side="right") - 1)
        for w in range(_NW)]
_R1S = [int(np.searchsorted(_OFF_NP, _A0S[w + 1] - 1, side="right"))
        for w in range(_NW)]
_WCLS = (2048, 1024, 512)
_CLS = [max(c for c, wdt in enumerate(_WCLS) if wdt >= 2047 - _R0S[w])
        for w in range(_NW)]


def _gen_body(labels_ref, thr_ref, ctab_ref, z_ref, w1_ref, b1_ref,
              w2_ref, b2_ref, we_ref, be_ref, out_ref, h_scr):
    i = pl.program_id(0)

    @pl.when(i == 0)
    def _():
        lab = labels_ref[0]
        sel = (lax.broadcasted_iota(jnp.int32, (_NUM_CLASSES, 1), 0) == lab)
        ce = jnp.sum(jnp.where(sel, ctab_ref[...], 0.0), axis=0, keepdims=True)
        b1_eff = b1_ref[...] + jnp.dot(ce, w1_ref[pl.ds(_NOISE_DIM, _CLASS_EMBED_DIM), :],
                                       preferred_element_type=jnp.float32)
        hgen = jnp.maximum(
            jnp.dot(z_ref[...], w1_ref[pl.ds(0, _NOISE_DIM), :],
                    preferred_element_type=jnp.float32) + b1_eff, 0.0)
        x = jnp.dot(hgen, w2_ref[...],
                    preferred_element_type=jnp.float32) + b2_ref[...]
        h_scr[...] = jnp.maximum(
            jnp.dot(x, we_ref[...],
                    preferred_element_type=jnp.float32) + be_ref[...], 0.0)

    hi = h_scr[pl.ds(i * _BLK, _BLK), :]
    h = h_scr[...]
    g = lax.dot_general(hi, h, (((1,), (1,)), ((), ())),
                        preferred_element_type=jnp.float32)
    sq_i = jnp.sum(hi * hi, axis=1, keepdims=True)
    sq_j = lax.dot_general(jnp.ones((1, _HIDDEN_DIM), jnp.float32), h * h,
                           (((1,), (1,)), ((), ())),
                           preferred_element_type=jnp.float32)
    d2 = sq_i + sq_j - 2.0 * g
    dist = jnp.sqrt(jnp.clip(d2, 1e-12, None))
    probs = jax.nn.sigmoid(thr_ref[0] - dist)
    rows = i * _BLK + lax.broadcasted_iota(jnp.int32, (_BLK, _N), 0)
    cols = lax.broadcasted_iota(jnp.int32, (_BLK, _N), 1)
    out_ref[...] = jnp.where(rows == cols, 0.0, probs)


def _tri_off(i):
    return (i * (2 * _N - 1 - i)) // 2


@functools.cache
def _make_triu_gather():
    @functools.partial(
        pl.kernel,
        out_type=jax.ShapeDtypeStruct((_M,), jnp.float32),
        mesh=plsc.VectorSubcoreMesh(core_axis_name="c", subcore_axis_name="s"),
        scratch_types=[
            pltpu.VMEM((2 * _KBUF * _ROWP,), jnp.float32),
            pltpu.VMEM((_SEG_MAX + _ROWP,), jnp.float32),
            [pltpu.SemaphoreType.DMA] * (2 * _KBUF),
        ],
    )
    def _triu_gather(adj_hbm, out_hbm, rows_v, seg_v, sems):
        cid = lax.axis_index("c")
        sid = lax.axis_index("s")
        wid = sid * 2 + cid

        a0 = jnp.int32(_A0S[0])
        seg_len = jnp.int32(_SEGL[0])
        r0 = jnp.int32(_R0S[0])
        r1 = jnp.int32(_R1S[0])
        cls = jnp.int32(_CLS[0])
        for w in range(1, _NW):
            a0 = jnp.where(wid == w, jnp.int32(_A0S[w]), a0)
            seg_len = jnp.where(wid == w, jnp.int32(_SEGL[w]), seg_len)
            r0 = jnp.where(wid == w, jnp.int32(_R0S[w]), r0)
            r1 = jnp.where(wid == w, jnp.int32(_R1S[w]), r1)
            cls = jnp.where(wid == w, jnp.int32(_CLS[w]), cls)
        a0 = pl.multiple_of(a0, 8)
        seg_len = pl.multiple_of(seg_len, 8)
        nrows = r1 - r0
        wsel = jnp.int32(_WCLS[0])
        for c in range(1, len(_WCLS)):
            wsel = jnp.where(cls == c, jnp.int32(_WCLS[c]), wsel)
        ngroups = (nrows + _KBUF - 1) // _KBUF

        def fire(t, bank):
            rbase = r0 + t * _KBUF
            for b in range(_KBUF):
                i = rbase + b
                slot = bank * _KBUF + b
                live = (t < ngroups) & (i < r1)
                for c, wdt in enumerate(_WCLS):

                    @pl.when(live & (cls == c))
                    def _(i=i, slot=slot, wdt=wdt):
                        pltpu.async_copy(
                            adj_hbm.at[pl.ds(i * _N + (_N - wdt), wdt)],
                            rows_v.at[pl.ds(slot * _ROWP, wdt)],
                            sems[slot])

        def process(t, bank):
            rbase = r0 + t * _KBUF
            for b in range(_KBUF):
                i = rbase + b
                slot = bank * _KBUF + b
                live = (t < ngroups) & (i < r1)
                for c, wdt in enumerate(_WCLS):

                    @pl.when(live & (cls == c))
                    def _(i=i, slot=slot, wdt=wdt):
                        pltpu.make_async_copy(
                            adj_hbm.at[pl.ds(i * _N + (_N - wdt), wdt)],
                            rows_v.at[pl.ds(slot * _ROWP, wdt)],
                            sems[slot]).wait()

                @pl.when(live)
                def _(i=i, slot=slot):
                    off_i = _tri_off(i)
                    skip = jnp.maximum(a0 - off_i, 0)
                    col0 = i + 1 + skip
                    q = off_i + skip - a0
                    length = (_N - 1 - i) - skip
                    nv = (length + 15) >> 4
                    src0 = slot * _ROWP + col0 - _N + wsel

                    def copy16(u, carry):
                        seg_v[pl.ds(q + u * 16, 16)] = (
                            rows_v[pl.ds(src0 + u * 16, 16)])
                        return carry

                    lax.fori_loop(0, nv, copy16, 0)

        fire(0, 0)

        def two_groups(tt, carry):
            t0 = 2 * tt
            fire(t0 + 1, 1)
            process(t0, 0)
            fire(t0 + 2, 0)
            process(t0 + 1, 1)
            return carry

        lax.fori_loop(0, (ngroups + 1) // 2, two_groups, 0)
        _bits = range(16, 2, -1)
        pos = jnp.int32(0)
        for n, p in enumerate(_bits):
            on = ((seg_len >> p) & 1) == 1

            @pl.when(on)
            def _(p=p, pos=pos, n=n):
                src = pl.multiple_of(pos, 8)
                dst = pl.multiple_of(a0 + pos, 8)
                pltpu.async_copy(seg_v.at[pl.ds(src, 1 << p)],
                                 out_hbm.at[pl.ds(dst, 1 << p)],
                                 sems[n])

            pos = pos + jnp.where(on, jnp.int32(1 << p), 0)
        pos = jnp.int32(0)
        for n, p in enumerate(_bits):
            on = ((seg_len >> p) & 1) == 1

            @pl.when(on)
            def _(p=p, pos=pos, n=n):
                src = pl.multiple_of(pos, 8)
                dst = pl.multiple_of(a0 + pos, 8)
                pltpu.make_async_copy(seg_v.at[pl.ds(src, 1 << p)],
                                      out_hbm.at[pl.ds(dst, 1 << p)],
                                      sems[n]).wait()

            pos = pos + jnp.where(on, jnp.int32(1 << p), 0)

    return _triu_gather


def kernel(class_labels, z, class_table, Wg1, bg1, Wg2, bg2, We, be, threshold):
    nblk = _N // _BLK
    soft_adj = pl.pallas_call(
        _gen_body,
        grid=(nblk,),
        out_shape=jax.ShapeDtypeStruct((_N, _N), jnp.float32),
        in_specs=[
            pl.BlockSpec(memory_space=pltpu.SMEM),
            pl.BlockSpec(memory_space=pltpu.SMEM),
            pl.BlockSpec(memory_space=pltpu.VMEM),
            pl.BlockSpec(memory_space=pltpu.VMEM),
            pl.BlockSpec(memory_space=pltpu.VMEM),
            pl.BlockSpec(memory_space=pltpu.VMEM),
            pl.BlockSpec(memory_space=pltpu.VMEM),
            pl.BlockSpec(memory_space=pltpu.VMEM),
            pl.BlockSpec(memory_space=pltpu.VMEM),
            pl.BlockSpec(memory_space=pltpu.VMEM),
        ],
        out_specs=pl.BlockSpec((_BLK, _N), lambda i: (i, 0)),
        scratch_shapes=[pltpu.VMEM((_N, _HIDDEN_DIM), jnp.float32)],
    )(class_labels, jnp.reshape(threshold, (1,)), class_table, z, Wg1,
      bg1[None, :], Wg2, bg2[None, :], We, be[None, :])

    probs_flat = _make_triu_gather()(jnp.reshape(soft_adj, (_N * _N,)))
    probs = probs_flat[:, None]

    pair_cols = _M // nblk
    pair_index = pl.pallas_call(
        _pair_copy_body,
        grid=(nblk,),
        out_shape=jax.ShapeDtypeStruct((2, _M), jnp.int32),
        in_specs=[pl.BlockSpec((2, pair_cols), lambda i: (0, i))],
        out_specs=pl.BlockSpec((2, pair_cols), lambda i: (0, i)),
    )(jnp.asarray(_PAIR_NP))
    return probs, pair_index, soft_adj
